# baseline (speedup 1.0000x reference)
import functools

import jax
import jax.numpy as jnp
from jax import lax
from jax.experimental import pallas as pl
from jax.experimental.pallas import tpu as pltpu
from jax.experimental.pallas import tpu_sc as plsc

VOCAB = 1000000
EMBED_DIM = 128
CTX = 16384

NC = 2
NS = 16
NW = NC * NS
IDX_PER_W = CTX // NW
CHUNK = 128
NCHUNK = IDX_PER_W // CHUNK
NLANE = EMBED_DIM // 16

_mesh = plsc.VectorSubcoreMesh(core_axis_name="c", subcore_axis_name="s")


@functools.partial(
    pl.kernel,
    mesh=_mesh,
    out_type=jax.ShapeDtypeStruct((NW, EMBED_DIM), jnp.float32),
    scratch_types=[
        pltpu.VMEM((NCHUNK, CHUNK), jnp.int32),
        pltpu.VMEM((NCHUNK, CHUNK, EMBED_DIM), jnp.float32),
        pltpu.VMEM((EMBED_DIM,), jnp.float32),
        pltpu.SemaphoreType.DMA,
    ],
)
def _gather_sum(idx_hbm, table_hbm, out_hbm, idx_v, rows_v, acc_v, sem):
    wid = lax.axis_index("s") * NC + lax.axis_index("c")
    pltpu.sync_copy(idx_hbm.at[wid], idx_v)
    copies = [
        pltpu.async_copy(table_hbm.at[idx_v.at[j]], rows_v.at[j], sem)
        for j in range(NCHUNK)
    ]
    acc = tuple(jnp.zeros((16,), jnp.float32) for _ in range(NLANE))
    for j in range(NCHUNK):
        copies[j].wait()

        def body(r, carry, j=j):
            return tuple(
                carry[c] + rows_v[j, r, c * 16:(c + 1) * 16]
                for c in range(NLANE)
            )
        acc = lax.fori_loop(0, CHUNK, body, acc)
    for c in range(NLANE):
        acc_v[c * 16:(c + 1) * 16] = acc[c]
    pltpu.sync_copy(acc_v, out_hbm.at[wid])


BLOCK_V = 16384
GRID_V = (VOCAB + BLOCK_V - 1) // BLOCK_V
DEPTH = 4


def _matvec_kernel(p_ref, b_ref, w_hbm, o_ref, ring, sems):
    hidden = jnp.sum(p_ref[...], axis=0, keepdims=True) * (1.0 / CTX)
    dn = (((1,), (1,)), ((), ()))

    def rows_at(j):
        return min(BLOCK_V, VOCAB - j * BLOCK_V)

    def issue(j):
        n = rows_at(j)
        return pltpu.async_copy(
            w_hbm.at[pl.ds(j * BLOCK_V, n), :],
            ring.at[j % DEPTH].at[pl.ds(0, n), :],
            sems.at[j % DEPTH],
        )

    handles = [issue(j) for j in range(DEPTH)]
    for j in range(GRID_V):
        handles[j].wait()
        n = rows_at(j)
        acc = lax.dot_general(
            hidden, ring[j % DEPTH, 0:n, :], dn,
            preferred_element_type=jnp.float32,
        )
        o_ref[0:1, j * BLOCK_V:j * BLOCK_V + n] = (
            acc + b_ref[0:1, j * BLOCK_V:j * BLOCK_V + n])
        if j + DEPTH < GRID_V:
            handles.append(issue(j + DEPTH))


_matvec = pl.pallas_call(
    _matvec_kernel,
    in_specs=[
        pl.BlockSpec(memory_space=pltpu.MemorySpace.VMEM),
        pl.BlockSpec(memory_space=pltpu.MemorySpace.VMEM),
        pl.BlockSpec(memory_space=pltpu.MemorySpace.HBM),
    ],
    out_specs=pl.BlockSpec(memory_space=pltpu.MemorySpace.VMEM),
    out_shape=jax.ShapeDtypeStruct((1, VOCAB), jnp.float32),
    scratch_shapes=[
        pltpu.VMEM((DEPTH, BLOCK_V, EMBED_DIM), jnp.float32),
        pltpu.SemaphoreType.DMA((DEPTH,)),
    ],
)


def kernel(context_indices, in_embeddings, out_weight, out_bias):
    idx3 = context_indices.reshape(NW, NCHUNK, CHUNK)
    partials = _gather_sum(idx3, in_embeddings)
    return _matvec(partials, out_bias.reshape(1, VOCAB), out_weight)


# speedup vs baseline: 1.0183x; 1.0183x over previous
import functools

import jax
import jax.numpy as jnp
from jax import lax
from jax.experimental import pallas as pl
from jax.experimental.pallas import tpu as pltpu
from jax.experimental.pallas import tpu_sc as plsc

VOCAB = 1000000
EMBED_DIM = 128
CTX = 16384

NC = 2
NS = 16
NW = NC * NS
IDX_PER_W = CTX // NW
CHUNK = 128
NCHUNK = IDX_PER_W // CHUNK
NLANE = EMBED_DIM // 16

_mesh = plsc.VectorSubcoreMesh(core_axis_name="c", subcore_axis_name="s")


@functools.partial(
    pl.kernel,
    mesh=_mesh,
    out_type=jax.ShapeDtypeStruct((NW, EMBED_DIM), jnp.float32),
    scratch_types=[
        pltpu.VMEM((NCHUNK, CHUNK), jnp.int32),
        pltpu.VMEM((NCHUNK, CHUNK, EMBED_DIM), jnp.float32),
        pltpu.VMEM((EMBED_DIM,), jnp.float32),
        pltpu.SemaphoreType.DMA,
    ],
)
def _gather_sum(idx_hbm, table_hbm, out_hbm, idx_v, rows_v, acc_v, sem):
    wid = lax.axis_index("s") * NC + lax.axis_index("c")
    pltpu.sync_copy(idx_hbm.at[wid], idx_v)
    copies = [
        pltpu.async_copy(table_hbm.at[idx_v.at[j]], rows_v.at[j], sem)
        for j in range(NCHUNK)
    ]
    acc = tuple(jnp.zeros((16,), jnp.float32) for _ in range(NLANE))
    for j in range(NCHUNK):
        copies[j].wait()

        def body(r, carry, j=j):
            return tuple(
                carry[c] + rows_v[j, r, c * 16:(c + 1) * 16]
                for c in range(NLANE)
            )
        acc = lax.fori_loop(0, CHUNK, body, acc)
    for c in range(NLANE):
        acc_v[c * 16:(c + 1) * 16] = acc[c]
    pltpu.sync_copy(acc_v, out_hbm.at[wid])


BLOCK_V = 16384
GRID_V = (VOCAB + BLOCK_V - 1) // BLOCK_V


def _matvec_kernel(p_ref, w_ref, b_ref, o_ref):
    hidden = jnp.sum(p_ref[...], axis=0, keepdims=True) * (1.0 / CTX)
    acc = lax.dot_general(
        hidden, w_ref[...], (((1,), (1,)), ((), ())),
        preferred_element_type=jnp.float32,
    )
    o_ref[...] = acc + b_ref[...]


_matvec = pl.pallas_call(
    _matvec_kernel,
    grid=(GRID_V,),
    in_specs=[
        pl.BlockSpec((NW, EMBED_DIM), lambda i: (0, 0)),
        pl.BlockSpec((BLOCK_V, EMBED_DIM), lambda i: (i, 0)),
        pl.BlockSpec((1, BLOCK_V), lambda i: (0, i)),
    ],
    out_specs=pl.BlockSpec((1, BLOCK_V), lambda i: (0, i)),
    out_shape=jax.ShapeDtypeStruct((1, VOCAB), jnp.float32),
)


def kernel(context_indices, in_embeddings, out_weight, out_bias):
    idx3 = context_indices.reshape(NW, NCHUNK, CHUNK)
    partials = _gather_sum(idx3, in_embeddings)
    return _matvec(partials, out_weight, out_bias.reshape(1, VOCAB))
